# baseline (speedup 1.0000x reference)
import functools
import math

import jax
import jax.numpy as jnp
from jax.experimental import pallas as pl
from jax.experimental.pallas import tpu as pltpu

_INV_SQRT2 = 1.0 / math.sqrt(2.0)


def _moe_block(x_ref, wr_ref, br_ref, w1_ref, b1_ref, w2_ref, b2_ref, o_ref):
    x = x_ref[...]
    logits = (
        jnp.dot(x, wr_ref[...], preferred_element_type=jnp.float32)
        + br_ref[...]
    )
    m = jnp.max(logits, axis=-1, keepdims=True)
    e = jnp.exp(logits - m)
    router = e / jnp.sum(e, axis=-1, keepdims=True)
    h = (
        jnp.dot(router, w1_ref[...], preferred_element_type=jnp.float32)
        + b1_ref[...]
    )
    g = h * 0.5 * (1.0 + jax.lax.erf(h * _INV_SQRT2))
    o_ref[...] = (
        jnp.dot(g, w2_ref[...], preferred_element_type=jnp.float32)
        + b2_ref[...]
    )


@functools.partial(jax.jit, static_argnames=("block_tokens", "interpret"))
def _moe_fused(x2d, Wr, br, W1l, b1l, W2l, b2l, block_tokens, interpret=False):
    n_tok, dim = x2d.shape
    e = Wr.shape[1]
    inner = W1l.shape[1]
    grid = (n_tok // block_tokens,)
    full = lambda shape: pl.BlockSpec(shape, lambda i: (0,) * len(shape))
    return pl.pallas_call(
        _moe_block,
        grid=grid,
        in_specs=[
            pl.BlockSpec((block_tokens, dim), lambda i: (i, 0)),
            full((dim, e)),
            full((1, e)),
            full((e, inner)),
            full((1, inner)),
            full((inner, dim)),
            full((1, dim)),
        ],
        out_specs=pl.BlockSpec((block_tokens, dim), lambda i: (i, 0)),
        out_shape=jax.ShapeDtypeStruct((n_tok, dim), x2d.dtype),
        compiler_params=pltpu.CompilerParams(
            dimension_semantics=("arbitrary",),
        ),
        interpret=interpret,
    )(x2d, Wr, br, W1l, b1l, W2l, b2l)


def kernel(x, Wr, br, W1, b1, W2, b2):
    B, S, DIM = x.shape
    x2d = x.reshape(B * S, DIM)
    out = _moe_fused(
        x2d,
        Wr,
        br.reshape(1, -1),
        W1[-1],
        b1[-1].reshape(1, -1),
        W2[-1],
        b2[-1].reshape(1, -1),
        block_tokens=2048,
    )
    return out.reshape(B, S, DIM)
